# baseline (speedup 1.0000x reference)
import functools

import jax
import jax.numpy as jnp
from jax import lax
from jax.experimental import pallas as pl
from jax.experimental.pallas import tpu as pltpu
from jax.experimental.pallas import tpu_sc as plsc

N = 10000
E = 320000
D = 128
NC = 2
NS = 16
NW = NC * NS
CHUNK = 64
N_CHUNKS = 5120
E_PAD = N_CHUNKS * CHUNK - E
CPW = N_CHUNKS // NW
SLAB = 40
N_PH = CPW // SLAB
RING = 4
N_TRASH = 112
ACC_ROWS = N + N_TRASH
ZSTRIPE = ACC_ROWS // NS
DSTRIPE = 640
BM = 1000


def _make_sc_aggregate():
    mesh = plsc.VectorSubcoreMesh(core_axis_name="c", subcore_axis_name="s")

    @functools.partial(
        pl.kernel,
        out_type=jax.ShapeDtypeStruct((NC, N, D), jnp.float32),
        mesh=mesh,
        scratch_types=[
            pltpu.VMEM((SLAB, CHUNK), jnp.int32),
            pltpu.VMEM((SLAB, CHUNK), jnp.int32),
            pltpu.VMEM((RING, CHUNK, D), jnp.float32),
            pltpu.VMEM_SHARED((ACC_ROWS, D), jnp.float32),
            pltpu.SemaphoreType.DMA,
            pltpu.SemaphoreType.DMA,
            pltpu.SemaphoreType.DMA,
            pltpu.SemaphoreType.DMA,
        ],
    )
    def sc_aggregate(col_hbm, row_hbm, x_hbm, zeros_hbm, out_hbm,
                     colv, rowv, gath, acc, sem_i, sem_g, sem_s, sem_z):
        c = lax.axis_index("c")
        s = lax.axis_index("s")
        wid = s * NC + c

        zd = pltpu.async_copy(zeros_hbm.at[pl.ds(s * ZSTRIPE, ZSTRIPE), :],
                              acc.at[pl.ds(s * ZSTRIPE, ZSTRIPE), :], sem_z)
        i0 = pltpu.async_copy(col_hbm.at[pl.ds(wid * CPW, SLAB), :], colv, sem_i)
        i1 = pltpu.async_copy(row_hbm.at[pl.ds(wid * CPW, SLAB), :], rowv, sem_i)
        zd.wait()
        plsc.subcore_barrier()

        for ph in range(N_PH):
            if ph == 0:
                i0.wait()
                i1.wait()
            else:
                base = wid * CPW + ph * SLAB
                pltpu.async_copy(col_hbm.at[pl.ds(base, SLAB), :], colv, sem_i).wait()
                pltpu.async_copy(row_hbm.at[pl.ds(base, SLAB), :], rowv, sem_i).wait()

            for b in range(RING):
                pltpu.async_copy(x_hbm.at[colv.at[b]], gath.at[b], sem_g)

            @pl.loop(0, SLAB, step=RING)
            def _(t):
                sds = []
                for b in range(RING):
                    tb = t + b
                    pltpu.make_async_copy(x_hbm.at[colv.at[tb]], gath.at[b],
                                          sem_g).wait()
                    sds.append(pltpu.async_copy(gath.at[b], acc.at[rowv.at[tb]],
                                                sem_s, add=True))
                for b in range(RING):
                    tb = t + b
                    sds[b].wait()

                    @pl.when(tb + RING < SLAB)
                    def _():
                        pltpu.async_copy(x_hbm.at[colv.at[tb + RING]],
                                         gath.at[b], sem_g)

        plsc.subcore_barrier()

        @pl.when(s < NS - 1)
        def _():
            pltpu.async_copy(acc.at[pl.ds(s * DSTRIPE, DSTRIPE), :],
                             out_hbm.at[c, pl.ds(s * DSTRIPE, DSTRIPE), :],
                             sem_z).wait()

        @pl.when(s == NS - 1)
        def _():
            tail = N - (NS - 1) * DSTRIPE
            pltpu.async_copy(acc.at[pl.ds((NS - 1) * DSTRIPE, tail), :],
                             out_hbm.at[c, pl.ds((NS - 1) * DSTRIPE, tail), :],
                             sem_z).wait()

    return sc_aggregate


_sc_aggregate = _make_sc_aggregate()


def _tc_matmul_body(p_ref, w_ref, b_ref, o_ref):
    agg = p_ref[0] + p_ref[1]
    o_ref[...] = (
        jnp.dot(agg, w_ref[...], preferred_element_type=jnp.float32)
        + b_ref[...]
    )


def _tc_matmul(partials, w, b):
    return pl.pallas_call(
        _tc_matmul_body,
        grid=(N // BM,),
        in_specs=[
            pl.BlockSpec((NC, BM, D), lambda i: (0, i, 0)),
            pl.BlockSpec((D, D), lambda i: (0, 0)),
            pl.BlockSpec((1, D), lambda i: (0, 0)),
        ],
        out_specs=pl.BlockSpec((BM, D), lambda i: (i, 0)),
        out_shape=jax.ShapeDtypeStruct((N, D), jnp.float32),
    )(partials, w, b)


def kernel(X, A_edge_index, W, b):
    idx = jnp.arange(E_PAD, dtype=jnp.int32)
    pad_row = N + idx % N_TRASH
    pad_col = (idx * 131) % N
    row = jnp.concatenate([A_edge_index[0], pad_row]).reshape(N_CHUNKS, CHUNK)
    col = jnp.concatenate([A_edge_index[1], pad_col]).reshape(N_CHUNKS, CHUNK)
    zeros = jnp.zeros((ACC_ROWS, D), jnp.float32)
    partials = _sc_aggregate(col, row, X, zeros)
    return _tc_matmul(partials, W, b.reshape(1, D))


# speedup vs baseline: 1.0318x; 1.0318x over previous
import functools

import jax
import jax.numpy as jnp
from jax import lax
from jax.experimental import pallas as pl
from jax.experimental.pallas import tpu as pltpu
from jax.experimental.pallas import tpu_sc as plsc

N = 10000
E = 320000
D = 128
NC = 2
NS = 16
NW = NC * NS
CHUNK = 128
N_CHUNKS = 2560
E_PAD = N_CHUNKS * CHUNK - E
CPW = N_CHUNKS // NW
SLAB = 40
N_PH = CPW // SLAB
RING = 2
N_TRASH = 112
ACC_ROWS = N + N_TRASH
ZSTRIPE = ACC_ROWS // NS
DSTRIPE = 640
BM = 1000


def _make_sc_aggregate():
    mesh = plsc.VectorSubcoreMesh(core_axis_name="c", subcore_axis_name="s")

    @functools.partial(
        pl.kernel,
        out_type=jax.ShapeDtypeStruct((NC, N, D), jnp.float32),
        mesh=mesh,
        scratch_types=[
            pltpu.VMEM((SLAB, CHUNK), jnp.int32),
            pltpu.VMEM((SLAB, CHUNK), jnp.int32),
            pltpu.VMEM((RING, CHUNK, D), jnp.float32),
            pltpu.VMEM_SHARED((ACC_ROWS, D), jnp.float32),
            pltpu.SemaphoreType.DMA,
            pltpu.SemaphoreType.DMA,
            pltpu.SemaphoreType.DMA,
            pltpu.SemaphoreType.DMA,
        ],
    )
    def sc_aggregate(col_hbm, row_hbm, x_hbm, zeros_hbm, out_hbm,
                     colv, rowv, gath, acc, sem_i, sem_g, sem_s, sem_z):
        c = lax.axis_index("c")
        s = lax.axis_index("s")
        wid = s * NC + c

        zd = pltpu.async_copy(zeros_hbm.at[pl.ds(s * ZSTRIPE, ZSTRIPE), :],
                              acc.at[pl.ds(s * ZSTRIPE, ZSTRIPE), :], sem_z)
        i0 = pltpu.async_copy(col_hbm.at[pl.ds(wid * CPW, SLAB), :], colv, sem_i)
        i1 = pltpu.async_copy(row_hbm.at[pl.ds(wid * CPW, SLAB), :], rowv, sem_i)
        zd.wait()
        plsc.subcore_barrier()

        for ph in range(N_PH):
            if ph == 0:
                i0.wait()
                i1.wait()
            else:
                base = wid * CPW + ph * SLAB
                pltpu.async_copy(col_hbm.at[pl.ds(base, SLAB), :], colv, sem_i).wait()
                pltpu.async_copy(row_hbm.at[pl.ds(base, SLAB), :], rowv, sem_i).wait()

            for b in range(RING):
                pltpu.async_copy(x_hbm.at[colv.at[b]], gath.at[b], sem_g)

            @pl.loop(0, SLAB, step=RING)
            def _(t):
                sds = []
                for b in range(RING):
                    tb = t + b
                    pltpu.make_async_copy(x_hbm.at[colv.at[tb]], gath.at[b],
                                          sem_g).wait()
                    sds.append(pltpu.async_copy(gath.at[b], acc.at[rowv.at[tb]],
                                                sem_s, add=True))
                for b in range(RING):
                    tb = t + b
                    sds[b].wait()

                    @pl.when(tb + RING < SLAB)
                    def _():
                        pltpu.async_copy(x_hbm.at[colv.at[tb + RING]],
                                         gath.at[b], sem_g)

        plsc.subcore_barrier()

        @pl.when(s < NS - 1)
        def _():
            pltpu.async_copy(acc.at[pl.ds(s * DSTRIPE, DSTRIPE), :],
                             out_hbm.at[c, pl.ds(s * DSTRIPE, DSTRIPE), :],
                             sem_z).wait()

        @pl.when(s == NS - 1)
        def _():
            tail = N - (NS - 1) * DSTRIPE
            pltpu.async_copy(acc.at[pl.ds((NS - 1) * DSTRIPE, tail), :],
                             out_hbm.at[c, pl.ds((NS - 1) * DSTRIPE, tail), :],
                             sem_z).wait()

    return sc_aggregate


_sc_aggregate = _make_sc_aggregate()


def _tc_matmul_body(p_ref, w_ref, b_ref, o_ref):
    agg = p_ref[0] + p_ref[1]
    o_ref[...] = (
        jnp.dot(agg, w_ref[...], preferred_element_type=jnp.float32)
        + b_ref[...]
    )


def _tc_matmul(partials, w, b):
    return pl.pallas_call(
        _tc_matmul_body,
        grid=(N // BM,),
        in_specs=[
            pl.BlockSpec((NC, BM, D), lambda i: (0, i, 0)),
            pl.BlockSpec((D, D), lambda i: (0, 0)),
            pl.BlockSpec((1, D), lambda i: (0, 0)),
        ],
        out_specs=pl.BlockSpec((BM, D), lambda i: (i, 0)),
        out_shape=jax.ShapeDtypeStruct((N, D), jnp.float32),
    )(partials, w, b)


def kernel(X, A_edge_index, W, b):
    idx = jnp.arange(E_PAD, dtype=jnp.int32)
    pad_row = N + idx % N_TRASH
    pad_col = (idx * 131) % N
    row = jnp.concatenate([A_edge_index[0], pad_row]).reshape(N_CHUNKS, CHUNK)
    col = jnp.concatenate([A_edge_index[1], pad_col]).reshape(N_CHUNKS, CHUNK)
    zeros = jnp.zeros((ACC_ROWS, D), jnp.float32)
    partials = _sc_aggregate(col, row, X, zeros)
    return _tc_matmul(partials, W, b.reshape(1, D))


# speedup vs baseline: 1.0671x; 1.0342x over previous
import functools

import jax
import jax.numpy as jnp
from jax import lax
from jax.experimental import pallas as pl
from jax.experimental.pallas import tpu as pltpu
from jax.experimental.pallas import tpu_sc as plsc

N = 10000
E = 320000
D = 128
NC = 2
NS = 16
NW = NC * NS
CHUNK = 128
N_CHUNKS = 2560
E_PAD = N_CHUNKS * CHUNK - E
CPW = N_CHUNKS // NW
SLAB = 40
N_TRASH = 240
ACC_ROWS = N + N_TRASH
ZSTRIPE = ACC_ROWS // NS
DSTRIPE = 640
BM = 1000


def _make_sc_aggregate():
    mesh = plsc.VectorSubcoreMesh(core_axis_name="c", subcore_axis_name="s")

    @functools.partial(
        pl.kernel,
        out_type=jax.ShapeDtypeStruct((NC, N, D), jnp.float32),
        mesh=mesh,
        scratch_types=[
            pltpu.VMEM((SLAB, CHUNK), jnp.int32),
            pltpu.VMEM((SLAB, CHUNK), jnp.int32),
            pltpu.VMEM((2, CHUNK, D), jnp.float32),
            pltpu.VMEM_SHARED((ACC_ROWS, D), jnp.float32),
            pltpu.SemaphoreType.DMA,
            pltpu.SemaphoreType.DMA,
            pltpu.SemaphoreType.DMA,
            pltpu.SemaphoreType.DMA,
        ],
    )
    def sc_aggregate(col_hbm, row_hbm, x_hbm, zeros_hbm, out_hbm,
                     colv, rowv, gath, acc, sem_i, sem_g, sem_s, sem_z):
        c = lax.axis_index("c")
        s = lax.axis_index("s")
        wid = s * NC + c

        zd = pltpu.async_copy(zeros_hbm.at[pl.ds(s * ZSTRIPE, ZSTRIPE), :],
                              acc.at[pl.ds(s * ZSTRIPE, ZSTRIPE), :], sem_z)
        i0 = pltpu.async_copy(col_hbm.at[pl.ds(wid * CPW, SLAB), :], colv, sem_i)
        i1 = pltpu.async_copy(row_hbm.at[pl.ds(wid * CPW, SLAB), :], rowv, sem_i)
        zd.wait()
        plsc.subcore_barrier()

        for ph in range(2):
            if ph == 0:
                i0.wait()
                i1.wait()
            else:
                base = wid * CPW + SLAB
                pltpu.async_copy(col_hbm.at[pl.ds(base, SLAB), :], colv, sem_i).wait()
                pltpu.async_copy(row_hbm.at[pl.ds(base, SLAB), :], rowv, sem_i).wait()

            pltpu.async_copy(x_hbm.at[colv.at[0]], gath.at[0], sem_g)
            pltpu.async_copy(x_hbm.at[colv.at[1]], gath.at[1], sem_g)

            @pl.loop(0, SLAB, step=2)
            def _(t):
                for b in range(2):
                    tb = t + b
                    pltpu.make_async_copy(x_hbm.at[colv.at[tb]], gath.at[b],
                                          sem_g).wait()
                    pltpu.async_copy(gath.at[b], acc.at[rowv.at[tb]],
                                     sem_s, add=True).wait()

                    @pl.when(tb + 2 < SLAB)
                    def _():
                        pltpu.async_copy(x_hbm.at[colv.at[tb + 2]],
                                         gath.at[b], sem_g)

        plsc.subcore_barrier()

        @pl.when(s < NS - 1)
        def _():
            pltpu.async_copy(acc.at[pl.ds(s * DSTRIPE, DSTRIPE), :],
                             out_hbm.at[c, pl.ds(s * DSTRIPE, DSTRIPE), :],
                             sem_z).wait()

        @pl.when(s == NS - 1)
        def _():
            tail = N - (NS - 1) * DSTRIPE
            pltpu.async_copy(acc.at[pl.ds((NS - 1) * DSTRIPE, tail), :],
                             out_hbm.at[c, pl.ds((NS - 1) * DSTRIPE, tail), :],
                             sem_z).wait()

    return sc_aggregate


_sc_aggregate = _make_sc_aggregate()


def _tc_matmul_body(p_ref, w_ref, b_ref, o_ref):
    agg = p_ref[0] + p_ref[1]
    o_ref[...] = (
        jnp.dot(agg, w_ref[...], preferred_element_type=jnp.float32)
        + b_ref[...]
    )


def _tc_matmul(partials, w, b):
    return pl.pallas_call(
        _tc_matmul_body,
        grid=(N // BM,),
        in_specs=[
            pl.BlockSpec((NC, BM, D), lambda i: (0, i, 0)),
            pl.BlockSpec((D, D), lambda i: (0, 0)),
            pl.BlockSpec((1, D), lambda i: (0, 0)),
        ],
        out_specs=pl.BlockSpec((BM, D), lambda i: (i, 0)),
        out_shape=jax.ShapeDtypeStruct((N, D), jnp.float32),
    )(partials, w, b)


def kernel(X, A_edge_index, W, b):
    idx = jnp.arange(E_PAD, dtype=jnp.int32)
    pad_row = N + idx % N_TRASH
    pad_col = (idx * 131) % N
    row = jnp.concatenate([A_edge_index[0], pad_row]).reshape(N_CHUNKS, CHUNK)
    col = jnp.concatenate([A_edge_index[1], pad_col]).reshape(N_CHUNKS, CHUNK)
    zeros = jnp.zeros((ACC_ROWS, D), jnp.float32)
    partials = _sc_aggregate(col, row, X, zeros)
    return _tc_matmul(partials, W, b.reshape(1, D))


# speedup vs baseline: 1.0871x; 1.0188x over previous
import functools

import jax
import jax.numpy as jnp
from jax import lax
from jax.experimental import pallas as pl
from jax.experimental.pallas import tpu as pltpu
from jax.experimental.pallas import tpu_sc as plsc

N = 10000
E = 320000
D = 128
NC = 2
NS = 16
NW = NC * NS
CHUNK = 128
N_CHUNKS = 2560
E_PAD = N_CHUNKS * CHUNK - E
CPW = N_CHUNKS // NW
SLAB = 40
N_TRASH = 240
ACC_ROWS = N + N_TRASH
ZSTRIPE = ACC_ROWS // NS
DSTRIPE = 640
BM = 2000


def _make_sc_aggregate():
    mesh = plsc.VectorSubcoreMesh(core_axis_name="c", subcore_axis_name="s")

    @functools.partial(
        pl.kernel,
        out_type=jax.ShapeDtypeStruct((NC, N, D), jnp.float32),
        mesh=mesh,
        scratch_types=[
            pltpu.VMEM((SLAB, CHUNK), jnp.int32),
            pltpu.VMEM((SLAB, CHUNK), jnp.int32),
            pltpu.VMEM((2, CHUNK, D), jnp.float32),
            pltpu.VMEM_SHARED((ACC_ROWS, D), jnp.float32),
            pltpu.SemaphoreType.DMA,
            pltpu.SemaphoreType.DMA,
            pltpu.SemaphoreType.DMA,
            pltpu.SemaphoreType.DMA,
        ],
    )
    def sc_aggregate(col_hbm, row_hbm, x_hbm, zeros_hbm, out_hbm,
                     colv, rowv, gath, acc, sem_i, sem_g, sem_s, sem_z):
        c = lax.axis_index("c")
        s = lax.axis_index("s")
        wid = s * NC + c

        zd = pltpu.async_copy(zeros_hbm.at[pl.ds(s * ZSTRIPE, ZSTRIPE), :],
                              acc.at[pl.ds(s * ZSTRIPE, ZSTRIPE), :], sem_z)
        i0 = pltpu.async_copy(col_hbm.at[pl.ds(wid * CPW, SLAB), :], colv, sem_i)
        i1 = pltpu.async_copy(row_hbm.at[pl.ds(wid * CPW, SLAB), :], rowv, sem_i)
        zd.wait()
        plsc.subcore_barrier()

        for ph in range(2):
            if ph == 0:
                i0.wait()
                i1.wait()
            else:
                base = wid * CPW + SLAB
                pltpu.async_copy(col_hbm.at[pl.ds(base, SLAB), :], colv, sem_i).wait()
                pltpu.async_copy(row_hbm.at[pl.ds(base, SLAB), :], rowv, sem_i).wait()

            pltpu.async_copy(x_hbm.at[colv.at[0]], gath.at[0], sem_g)
            pltpu.async_copy(x_hbm.at[colv.at[1]], gath.at[1], sem_g)

            @pl.loop(0, SLAB, step=2)
            def _(t):
                for b in range(2):
                    tb = t + b
                    pltpu.make_async_copy(x_hbm.at[colv.at[tb]], gath.at[b],
                                          sem_g).wait()
                    pltpu.async_copy(gath.at[b], acc.at[rowv.at[tb]],
                                     sem_s, add=True).wait()

                    @pl.when(tb + 2 < SLAB)
                    def _():
                        pltpu.async_copy(x_hbm.at[colv.at[tb + 2]],
                                         gath.at[b], sem_g)

        plsc.subcore_barrier()

        @pl.when(s < NS - 1)
        def _():
            pltpu.async_copy(acc.at[pl.ds(s * DSTRIPE, DSTRIPE), :],
                             out_hbm.at[c, pl.ds(s * DSTRIPE, DSTRIPE), :],
                             sem_z).wait()

        @pl.when(s == NS - 1)
        def _():
            tail = N - (NS - 1) * DSTRIPE
            pltpu.async_copy(acc.at[pl.ds((NS - 1) * DSTRIPE, tail), :],
                             out_hbm.at[c, pl.ds((NS - 1) * DSTRIPE, tail), :],
                             sem_z).wait()

    return sc_aggregate


_sc_aggregate = _make_sc_aggregate()


def _tc_matmul_body(p_ref, w_ref, b_ref, o_ref):
    agg = p_ref[0] + p_ref[1]
    o_ref[...] = (
        jnp.dot(agg, w_ref[...], preferred_element_type=jnp.float32)
        + b_ref[...]
    )


def _tc_matmul(partials, w, b):
    return pl.pallas_call(
        _tc_matmul_body,
        grid=(N // BM,),
        in_specs=[
            pl.BlockSpec((NC, BM, D), lambda i: (0, i, 0)),
            pl.BlockSpec((D, D), lambda i: (0, 0)),
            pl.BlockSpec((1, D), lambda i: (0, 0)),
        ],
        out_specs=pl.BlockSpec((BM, D), lambda i: (i, 0)),
        out_shape=jax.ShapeDtypeStruct((N, D), jnp.float32),
    )(partials, w, b)


def kernel(X, A_edge_index, W, b):
    idx = jnp.arange(E_PAD, dtype=jnp.int32)
    pad_row = N + idx % N_TRASH
    pad_col = (idx * 131) % N
    row = jnp.concatenate([A_edge_index[0], pad_row]).reshape(N_CHUNKS, CHUNK)
    col = jnp.concatenate([A_edge_index[1], pad_col]).reshape(N_CHUNKS, CHUNK)
    zeros = jnp.zeros((ACC_ROWS, D), jnp.float32)
    partials = _sc_aggregate(col, row, X, zeros)
    return _tc_matmul(partials, W, b.reshape(1, D))
